# baseline (speedup 1.0000x reference)
import functools

import jax
import jax.numpy as jnp
from jax import lax
from jax.experimental import pallas as pl
from jax.experimental.pallas import tpu as pltpu
from jax.experimental.pallas import tpu_sc as plsc

B = 16384
L = 50
D = 10
DP = 16

NC = 2
NS = 16
NW = NC * NS
NB = B // NW
CB = 64
NCHUNK = NB // CB
NG = CB * L // 128
GROUPS = CB // 16
VOCABS = (1000000, 1000000, 1000000, 100000, 100000, 100000)

C1 = (L - 1.0) / L
C2 = 1.0 / (L * L)


def _prep_body(tab_ref, w_ref, b_ref, out_ref):
    x = tab_ref[...]
    w = w_ref[...]
    out_ref[...] = lax.dot_general(
        x, w, (((1,), (0,)), ((), ())),
        preferred_element_type=jnp.float32) + b_ref[0, 0]


def _prep_table(tab, w, b):
    rows = tab.shape[0]
    blk = 25000
    grid = rows // blk
    s = pl.pallas_call(
        _prep_body,
        grid=(grid,),
        in_specs=[
            pl.BlockSpec((blk, D), lambda i: (i, 0)),
            pl.BlockSpec((D, 1), lambda i: (0, 0)),
            pl.BlockSpec((1, 1), lambda i: (0, 0)),
        ],
        out_specs=pl.BlockSpec((blk, 1), lambda i: (i, 0)),
        out_shape=jax.ShapeDtypeStruct((rows, 1), jnp.float32),
    )(tab, w, b.reshape(1, 1))
    z = jnp.zeros((rows, DP - D - 1), jnp.float32)
    return jnp.concatenate([tab, s, z], axis=1)


def _sc_body(kwtab, tgtab, kwidx, tgidx,
             fi0, fi1, fi2, fi3, fi4, fi5,
             ft0, ft1, ft2, ft3, ft4, ft5,
             kwout, tgout, fo0, fo1, fo2, fo3, fo4, fo5,
             ib, rb, out_v, fidx_v, if_v, frows_v, sem0, sem1):
    wid = lax.axis_index("s") * NC + lax.axis_index("c")
    base_b = wid * NB
    lane = jnp.arange(16, dtype=jnp.int32)
    sems = (sem0, sem1)
    needs = [min(NG, -(-((bi + 1) * 16 * L) // 128)) for bi in range(GROUPS)]

    def do_table(tab_ref, idx_ref, out_ref):
        idx0 = wid * (NB * L)

        def mkcps(p):
            return [pltpu.make_async_copy(
                tab_ref.at[ib.at[p].at[pl.ds(g * 128, 128)]],
                rb.at[p].at[pl.ds(g * 128, 128)], sems[p])
                for g in range(NG)]

        def stage(c, p):
            pltpu.sync_copy(idx_ref.at[pl.ds(idx0 + c * (CB * L), CB * L)],
                            ib.at[p])
            for cp in mkcps(p):
                cp.start()

        def consume(c, p):
            rbp = rb.at[p]
            cps = mkcps(p)
            done = 0
            for bi in range(GROUPS):
                for g in range(done, needs[bi]):
                    cps[g].wait()
                done = needs[bi]
                row_base = bi * 16 * L + lane * L

                def l_sum(l, ssum):
                    s = plsc.load_gather(
                        rbp, [row_base + l, jnp.full((16,), D, jnp.int32)])
                    return ssum + s
                ssum = lax.fori_loop(0, L, l_sum,
                                     jnp.zeros((16,), jnp.float32))

                def l_acc(l, accs):
                    r = row_base + l
                    s = plsc.load_gather(
                        rbp, [r, jnp.full((16,), D, jnp.int32)])
                    score = C1 * s + C2 * ssum
                    p_ = 1.0 / (1.0 + jnp.exp(-score))
                    return tuple(
                        accs[d] + p_ * plsc.load_gather(
                            rbp, [r, jnp.full((16,), d, jnp.int32)])
                        for d in range(D))
                accs = lax.fori_loop(
                    0, L, l_acc,
                    tuple(jnp.zeros((16,), jnp.float32) for _ in range(D)))
                for d in range(D):
                    plsc.store_scatter(
                        out_v, [bi * 16 + lane, jnp.full((16,), d, jnp.int32)],
                        accs[d])
            pltpu.sync_copy(out_v, out_ref.at[pl.ds(base_b + c * CB, CB)])

        stage(0, 0)

        def body(i, carry):
            c = 2 * i
            stage(c + 1, 1)
            consume(c, 0)

            @pl.when(i < NCHUNK // 2 - 1)
            def _():
                stage(c + 2, 0)
            consume(c + 1, 1)
            return carry
        lax.fori_loop(0, NCHUNK // 2, body, 0)

    do_table(kwtab, kwidx, kwout)
    do_table(tgtab, tgidx, tgout)

    for (fi, ft, fo), V in zip(
            ((fi0, ft0, fo0), (fi1, ft1, fo1), (fi2, ft2, fo2),
             (fi3, ft3, fo3), (fi4, ft4, fo4), (fi5, ft5, fo5)), VOCABS):
        pltpu.sync_copy(fi.at[pl.ds(base_b, NB)], fidx_v)

        def mkidx(g, carry):
            v = plsc.load_gather(fidx_v, [g * 16 + lane])
            pos = g * (16 * D) + lane * D
            for d in range(D):
                plsc.store_scatter(if_v, [pos + d], v + d * V)
            return carry
        lax.fori_loop(0, NB // 16, mkidx, 0)

        def fgather(j, carry):
            cps = []
            for k in range(8):
                sl = pl.ds(j * 1024 + k * 128, 128)
                cp = pltpu.make_async_copy(ft.at[if_v.at[sl]],
                                           frows_v.at[sl], sem0)
                cp.start()
                cps.append(cp)
            for cp in cps:
                cp.wait()
            return carry
        lax.fori_loop(0, NB * D // 1024, fgather, 0)
        pltpu.sync_copy(frows_v, fo.at[pl.ds(base_b * D, NB * D)])


_sc_call_cache = []


def _get_sc_call():
    if not _sc_call_cache:
        _sc_call_cache.append(functools.partial(
            pl.kernel,
            out_type=[jax.ShapeDtypeStruct((B, DP), jnp.float32),
                      jax.ShapeDtypeStruct((B, DP), jnp.float32)] +
                     [jax.ShapeDtypeStruct((B * D,), jnp.float32)] * 6,
            mesh=plsc.VectorSubcoreMesh(
                core_axis_name="c", subcore_axis_name="s",
                num_cores=NC, num_subcores=NS),
            compiler_params=pltpu.CompilerParams(
                needs_layout_passes=False, use_tc_tiling_on_sc=False),
            scratch_types=[
                pltpu.VMEM((2, CB * L), jnp.int32),
                pltpu.VMEM((2, CB * L, DP), jnp.float32),
                pltpu.VMEM((CB, DP), jnp.float32),
                pltpu.VMEM((NB,), jnp.int32),
                pltpu.VMEM((NB * D,), jnp.int32),
                pltpu.VMEM((NB * D,), jnp.float32),
                pltpu.SemaphoreType.DMA,
                pltpu.SemaphoreType.DMA,
            ],
        )(_sc_body))
    return _sc_call_cache[0]


def _final_body(kw_ref, tg_ref, f0, f1, f2, f3, f4, f5,
                dense_ref, tw_ref, w_ref, out_ref):
    dn = (((1,), (0,)), ((), ()))
    acc = lax.dot_general(kw_ref[:, 0:D], tw_ref[0:D, :], dn,
                          preferred_element_type=jnp.float32)
    acc += lax.dot_general(tg_ref[:, 0:D], tw_ref[D:2 * D, :], dn,
                           preferred_element_type=jnp.float32)
    for i, f in enumerate((f0, f1, f2, f3, f4, f5)):
        acc += lax.dot_general(f[...], tw_ref[(2 + i) * D:(3 + i) * D, :], dn,
                               preferred_element_type=jnp.float32)
    acc += dense_ref[...] * w_ref[...]
    out_ref[...] = acc


def _final(kw, tg, feats, dense, tw, w):
    blk = 2048
    grid = B // blk
    row_spec16 = pl.BlockSpec((blk, DP), lambda i: (i, 0))
    row_spec10 = pl.BlockSpec((blk, D), lambda i: (i, 0))
    return pl.pallas_call(
        _final_body,
        grid=(grid,),
        in_specs=[row_spec16, row_spec16] + [row_spec10] * 6 + [
            pl.BlockSpec((blk, 1), lambda i: (i, 0)),
            pl.BlockSpec((8 * D, 7), lambda i: (0, 0)),
            pl.BlockSpec((1, 7), lambda i: (0, 0)),
        ],
        out_specs=pl.BlockSpec((blk, 7), lambda i: (i, 0)),
        out_shape=jax.ShapeDtypeStruct((B, 7), jnp.float32),
    )(kw, tg, *feats, dense, tw, w)


def kernel(dense_features, keyword_idx, tag_idx, feat0_idx, feat1_idx,
           feat2_idx, feat3_idx, feat4_idx, feat5_idx, emb_keyword, emb_tag,
           emb_feat0, emb_feat1, emb_feat2, emb_feat3, emb_feat4, emb_feat5,
           attn_key_w, attn_key_b, attn_tag_w, attn_tag_b, weight,
           trans_weight):
    kw_pad = _prep_table(emb_keyword, attn_key_w, attn_key_b)
    tg_pad = _prep_table(emb_tag, attn_tag_w, attn_tag_b)

    kwidx = keyword_idx.reshape(-1)
    tgidx = tag_idx.reshape(-1)
    fidx = (feat0_idx, feat1_idx, feat2_idx, feat3_idx, feat4_idx, feat5_idx)
    ftabs = [t.T.reshape(-1) for t in (emb_feat0, emb_feat1, emb_feat2,
                                       emb_feat3, emb_feat4, emb_feat5)]

    kw_p, tg_p, *feats = _get_sc_call()(kw_pad, tg_pad, kwidx, tgidx,
                                        *fidx, *ftabs)
    feats = [f.reshape(B, D) for f in feats]

    return _final(kw_p, tg_p, feats, dense_features, trans_weight, weight)


# speedup vs baseline: 1.3314x; 1.3314x over previous
import functools

import jax
import jax.numpy as jnp
from jax import lax
from jax.experimental import pallas as pl
from jax.experimental.pallas import tpu as pltpu
from jax.experimental.pallas import tpu_sc as plsc

B = 16384
L = 50
D = 10
DP = 16

NC = 2
NS = 16
NW = NC * NS
NB = B // NW
CB = 64
NCHUNK = NB // CB
NG = CB * L // 128
GROUPS = CB // 16
VOCABS = (1000000, 1000000, 1000000, 100000, 100000, 100000)

C1 = (L - 1.0) / L
C2 = 1.0 / (L * L)


def _prep_body(tab_ref, w_ref, b_ref, out_ref):
    x = tab_ref[...]
    w = w_ref[...]
    out_ref[...] = lax.dot_general(
        x, w, (((1,), (0,)), ((), ())),
        preferred_element_type=jnp.float32) + b_ref[0, 0]


def _prep_table(tab, w, b):
    rows = tab.shape[0]
    blk = 25000
    grid = rows // blk
    s = pl.pallas_call(
        _prep_body,
        grid=(grid,),
        in_specs=[
            pl.BlockSpec((blk, D), lambda i: (i, 0)),
            pl.BlockSpec((D, 1), lambda i: (0, 0)),
            pl.BlockSpec((1, 1), lambda i: (0, 0)),
        ],
        out_specs=pl.BlockSpec((blk, 1), lambda i: (i, 0)),
        out_shape=jax.ShapeDtypeStruct((rows, 1), jnp.float32),
    )(tab, w, b.reshape(1, 1))
    z = jnp.zeros((rows, DP - D - 1), jnp.float32)
    return jnp.concatenate([tab, s, z], axis=1)


def _sc_body(kwtab, tgtab, kwidx, tgidx,
             fi0, fi1, fi2, fi3, fi4, fi5,
             ft0, ft1, ft2, ft3, ft4, ft5,
             kwout, tgout, fo0, fo1, fo2, fo3, fo4, fo5,
             ib, rb, out_v, fidx_v, if_v, frows_v, sem0, sem1):
    wid = lax.axis_index("s") * NC + lax.axis_index("c")
    base_b = wid * NB
    lane = jnp.arange(16, dtype=jnp.int32)
    sems = (sem0, sem1)
    needs = [min(NG, -(-((bi + 1) * 16 * L) // 128)) for bi in range(GROUPS)]

    def do_table(tab_ref, idx_ref, out_ref):
        idx0 = wid * (NB * L)

        def mkcps(p):
            return [pltpu.make_async_copy(
                tab_ref.at[ib.at[p].at[pl.ds(g * 128, 128)]],
                rb.at[p].at[pl.ds(g * 128, 128)], sems[p])
                for g in range(NG)]

        def stage(c, p):
            pltpu.sync_copy(idx_ref.at[pl.ds(idx0 + c * (CB * L), CB * L)],
                            ib.at[p])
            for cp in mkcps(p):
                cp.start()

        def consume(c, p):
            rbp = rb.at[p]
            cps = mkcps(p)
            done = 0
            for bi in range(GROUPS):
                for g in range(done, needs[bi]):
                    cps[g].wait()
                done = needs[bi]
                row_base = bi * 16 * L + lane * L

                def l_sum(l, ssum):
                    s = plsc.load_gather(
                        rbp, [row_base + l, jnp.full((16,), D, jnp.int32)])
                    return ssum + s
                ssum = lax.fori_loop(0, L, l_sum,
                                     jnp.zeros((16,), jnp.float32))

                def l_acc(l, accs):
                    r = row_base + l
                    s = plsc.load_gather(
                        rbp, [r, jnp.full((16,), D, jnp.int32)])
                    score = C1 * s + C2 * ssum
                    p_ = 1.0 / (1.0 + jnp.exp(-score))
                    return tuple(
                        accs[d] + p_ * plsc.load_gather(
                            rbp, [r, jnp.full((16,), d, jnp.int32)])
                        for d in range(D))
                accs = lax.fori_loop(
                    0, L, l_acc,
                    tuple(jnp.zeros((16,), jnp.float32) for _ in range(D)))
                for d in range(D):
                    plsc.store_scatter(
                        out_v, [bi * 16 + lane, jnp.full((16,), d, jnp.int32)],
                        accs[d])
            pltpu.sync_copy(out_v, out_ref.at[pl.ds(base_b + c * CB, CB)])

        stage(0, 0)

        def body(i, carry):
            c = 2 * i
            stage(c + 1, 1)
            consume(c, 0)

            @pl.when(i < NCHUNK // 2 - 1)
            def _():
                stage(c + 2, 0)
            consume(c + 1, 1)
            return carry
        lax.fori_loop(0, NCHUNK // 2, body, 0)

    do_table(kwtab, kwidx, kwout)
    do_table(tgtab, tgidx, tgout)

    for (fi, ft, fo), V in zip(
            ((fi0, ft0, fo0), (fi1, ft1, fo1), (fi2, ft2, fo2),
             (fi3, ft3, fo3), (fi4, ft4, fo4), (fi5, ft5, fo5)), VOCABS):
        pltpu.sync_copy(fi.at[pl.ds(base_b, NB)], fidx_v)

        def mkidx(g, carry):
            v = plsc.load_gather(fidx_v, [g * 16 + lane])
            pos = g * (16 * D) + lane * D
            v10 = v * D
            for d in range(D):
                plsc.store_scatter(if_v, [pos + d], v10 + d)
            return carry
        lax.fori_loop(0, NB // 16, mkidx, 0)

        def fgather(j, carry):
            cps = []
            for k in range(8):
                sl = pl.ds(j * 1024 + k * 128, 128)
                cp = pltpu.make_async_copy(ft.at[if_v.at[sl]],
                                           frows_v.at[sl], sem0)
                cp.start()
                cps.append(cp)
            for cp in cps:
                cp.wait()
            return carry
        lax.fori_loop(0, NB * D // 1024, fgather, 0)
        pltpu.sync_copy(frows_v, fo.at[pl.ds(base_b * D, NB * D)])


_sc_call_cache = []


def _get_sc_call():
    if not _sc_call_cache:
        _sc_call_cache.append(functools.partial(
            pl.kernel,
            out_type=[jax.ShapeDtypeStruct((B, DP), jnp.float32),
                      jax.ShapeDtypeStruct((B, DP), jnp.float32)] +
                     [jax.ShapeDtypeStruct((B * D,), jnp.float32)] * 6,
            mesh=plsc.VectorSubcoreMesh(
                core_axis_name="c", subcore_axis_name="s",
                num_cores=NC, num_subcores=NS),
            compiler_params=pltpu.CompilerParams(
                needs_layout_passes=False, use_tc_tiling_on_sc=False),
            scratch_types=[
                pltpu.VMEM((2, CB * L), jnp.int32),
                pltpu.VMEM((2, CB * L, DP), jnp.float32),
                pltpu.VMEM((CB, DP), jnp.float32),
                pltpu.VMEM((NB,), jnp.int32),
                pltpu.VMEM((NB * D,), jnp.int32),
                pltpu.VMEM((NB * D,), jnp.float32),
                pltpu.SemaphoreType.DMA,
                pltpu.SemaphoreType.DMA,
            ],
        )(_sc_body))
    return _sc_call_cache[0]


def _final_body(kw_ref, tg_ref, f0, f1, f2, f3, f4, f5,
                dense_ref, tw_ref, w_ref, out_ref):
    dn = (((1,), (0,)), ((), ()))
    acc = lax.dot_general(kw_ref[:, 0:D], tw_ref[0:D, :], dn,
                          preferred_element_type=jnp.float32)
    acc += lax.dot_general(tg_ref[:, 0:D], tw_ref[D:2 * D, :], dn,
                           preferred_element_type=jnp.float32)
    for i, f in enumerate((f0, f1, f2, f3, f4, f5)):
        acc += lax.dot_general(f[...], tw_ref[(2 + i) * D:(3 + i) * D, :], dn,
                               preferred_element_type=jnp.float32)
    acc += dense_ref[...] * w_ref[...]
    out_ref[...] = acc


def _final(kw, tg, feats, dense, tw, w):
    blk = 2048
    grid = B // blk
    row_spec16 = pl.BlockSpec((blk, DP), lambda i: (i, 0))
    row_spec10 = pl.BlockSpec((blk, D), lambda i: (i, 0))
    return pl.pallas_call(
        _final_body,
        grid=(grid,),
        in_specs=[row_spec16, row_spec16] + [row_spec10] * 6 + [
            pl.BlockSpec((blk, 1), lambda i: (i, 0)),
            pl.BlockSpec((8 * D, 7), lambda i: (0, 0)),
            pl.BlockSpec((1, 7), lambda i: (0, 0)),
        ],
        out_specs=pl.BlockSpec((blk, 7), lambda i: (i, 0)),
        out_shape=jax.ShapeDtypeStruct((B, 7), jnp.float32),
    )(kw, tg, *feats, dense, tw, w)


def _to_rowmajor(t):
    return jnp.concatenate([t[:, i:i + 1] for i in range(t.shape[1])], axis=1)


def kernel(dense_features, keyword_idx, tag_idx, feat0_idx, feat1_idx,
           feat2_idx, feat3_idx, feat4_idx, feat5_idx, emb_keyword, emb_tag,
           emb_feat0, emb_feat1, emb_feat2, emb_feat3, emb_feat4, emb_feat5,
           attn_key_w, attn_key_b, attn_tag_w, attn_tag_b, weight,
           trans_weight):
    kw_pad = _prep_table(_to_rowmajor(emb_keyword), attn_key_w, attn_key_b)
    tg_pad = _prep_table(_to_rowmajor(emb_tag), attn_tag_w, attn_tag_b)

    kwidx = keyword_idx.reshape(-1)
    tgidx = tag_idx.reshape(-1)
    fidx = (feat0_idx, feat1_idx, feat2_idx, feat3_idx, feat4_idx, feat5_idx)
    ftabs = [_to_rowmajor(t).reshape(-1)
             for t in (emb_feat0, emb_feat1, emb_feat2,
                       emb_feat3, emb_feat4, emb_feat5)]

    kw_p, tg_p, *feats = _get_sc_call()(kw_pad, tg_pad, kwidx, tgidx,
                                        *fidx, *ftabs)
    feats = [f.reshape(B, D) for f in feats]

    return _final(kw_p, tg_p, feats, dense_features, trans_weight, weight)
